# baseline (speedup 1.0000x reference)
import functools

import jax
import jax.numpy as jnp
from jax import lax
from jax.experimental import pallas as pl
from jax.experimental.pallas import tpu as pltpu
from jax.experimental.pallas import tpu_sc as plsc

EMBED = 64
BATCH = 16384
LANES = 16

_info = plsc.get_sparse_core_info()
NC = _info.num_cores
NS = _info.num_subcores
NW = NC * NS
B_PER_W = BATCH // NW
CHUNK = 128
NCHUNK = B_PER_W // CHUNK
NV = EMBED // LANES


def _take16(x, idx):
    return lax.gather(
        x,
        idx[:, None],
        dimension_numbers=lax.GatherDimensionNumbers(
            offset_dims=(), collapsed_slice_dims=(0,), start_index_map=(0,)
        ),
        slice_sizes=(1,),
        mode=lax.GatherScatterMode.PROMISE_IN_BOUNDS,
    )


def _rsqrt(x):
    i = lax.bitcast_convert_type(x, jnp.int32)
    i = jnp.int32(0x5F3759DF) - (i >> 1)
    y = lax.bitcast_convert_type(i, jnp.float32)
    for _ in range(3):
        y = y * (1.5 - 0.5 * x * y * y)
    return y


@functools.partial(
    pl.kernel,
    mesh=plsc.VectorSubcoreMesh(core_axis_name="c", subcore_axis_name="s"),
    out_type=jax.ShapeDtypeStruct((BATCH, EMBED), jnp.float32),
    scratch_types=[
        pltpu.VMEM((NCHUNK, CHUNK), jnp.int32),
        pltpu.VMEM((B_PER_W, EMBED), jnp.float32),
        pltpu.SemaphoreType.DMA,
    ],
    compiler_params=pltpu.CompilerParams(use_tc_tiling_on_sc=False),
)
def _embed_norm(table_hbm, idx_hbm, out_hbm, idx_v, rows_v, sem):
    wid = lax.axis_index("s") * NC + lax.axis_index("c")
    base = wid * B_PER_W

    pltpu.sync_copy(idx_hbm.at[wid], idx_v)

    copies = []
    for c in range(NCHUNK):
        copies.append(
            pltpu.async_copy(
                table_hbm.at[idx_v.at[c]],
                rows_v.at[pl.ds(c * CHUNK, CHUNK)],
                sem,
            )
        )
    for cp in copies:
        cp.wait()

    lanes = lax.iota(jnp.int32, LANES)

    def _norm_row(r, carry):
        vs = [rows_v[r, pl.ds(LANES * j, LANES)] for j in range(NV)]
        ssq = vs[0] * vs[0]
        for j in range(1, NV):
            ssq = ssq + vs[j] * vs[j]
        for sh in (8, 4, 2, 1):
            ssq = ssq + _take16(ssq, lanes ^ sh)
        y = _rsqrt(ssq + 1e-12)
        for j in range(NV):
            rows_v[r, pl.ds(LANES * j, LANES)] = vs[j] * y
        return carry

    lax.fori_loop(0, B_PER_W, _norm_row, 0)

    pltpu.sync_copy(rows_v, out_hbm.at[pl.ds(base, B_PER_W)])


def kernel(indices, table):
    idx = indices.astype(jnp.int32).reshape(NW, NCHUNK, CHUNK)
    return _embed_norm(table, idx)


# speedup vs baseline: 1.6518x; 1.6518x over previous
import functools

import jax
import jax.numpy as jnp
from jax import lax
from jax.experimental import pallas as pl
from jax.experimental.pallas import tpu as pltpu
from jax.experimental.pallas import tpu_sc as plsc

EMBED = 64
BATCH = 16384
LANES = 16

_info = plsc.get_sparse_core_info()
NC = _info.num_cores
NS = _info.num_subcores
NW = NC * NS
B_PER_W = BATCH // NW
GROUP = 16
NGROUP = B_PER_W // GROUP
NV = EMBED // LANES


def _take16(x, idx):
    return lax.gather(
        x,
        idx[:, None],
        dimension_numbers=lax.GatherDimensionNumbers(
            offset_dims=(), collapsed_slice_dims=(0,), start_index_map=(0,)
        ),
        slice_sizes=(1,),
        mode=lax.GatherScatterMode.PROMISE_IN_BOUNDS,
    )


def _rsqrt(x):
    i = lax.bitcast_convert_type(x, jnp.int32)
    i = jnp.int32(0x5F3759DF) - (i >> 1)
    y = lax.bitcast_convert_type(i, jnp.float32)
    for _ in range(3):
        y = y * (1.5 - 0.5 * x * y * y)
    return y


@functools.partial(
    pl.kernel,
    mesh=plsc.VectorSubcoreMesh(core_axis_name="c", subcore_axis_name="s"),
    out_type=jax.ShapeDtypeStruct((BATCH, EMBED), jnp.float32),
    scratch_types=[
        pltpu.VMEM((B_PER_W,), jnp.int32),
        pltpu.VMEM((B_PER_W, EMBED), jnp.float32),
        pltpu.SemaphoreType.DMA,
    ],
)
def _embed_norm(table_hbm, idx_hbm, out_hbm, idx_v, rows_v, sem):
    wid = lax.axis_index("s") * NC + lax.axis_index("c")
    base = wid * B_PER_W

    pltpu.sync_copy(idx_hbm.at[wid], idx_v)

    lanes = lax.iota(jnp.int32, LANES)

    def _group(g, carry):
        rb = g * GROUP
        vidx = idx_v[pl.ds(rb, GROUP)]
        cps = []
        for j in range(GROUP):
            row = vidx[j]
            cps.append(
                pltpu.async_copy(
                    table_hbm.at[pl.ds(row, 1), :],
                    rows_v.at[pl.ds(rb + j, 1), :],
                    sem,
                )
            )
        for cp in cps:
            cp.wait()
        for j in range(GROUP):
            r = rb + j
            vs = [rows_v[r, pl.ds(LANES * k, LANES)] for k in range(NV)]
            ssq = vs[0] * vs[0]
            for k in range(1, NV):
                ssq = ssq + vs[k] * vs[k]
            for sh in (8, 4, 2, 1):
                ssq = ssq + _take16(ssq, lanes ^ sh)
            y = _rsqrt(ssq + 1e-12)
            for k in range(NV):
                rows_v[r, pl.ds(LANES * k, LANES)] = vs[k] * y
        return carry

    lax.fori_loop(0, NGROUP, _group, 0)

    pltpu.sync_copy(rows_v, out_hbm.at[pl.ds(base, B_PER_W)])


def kernel(indices, table):
    idx = indices.astype(jnp.int32).reshape(NW, B_PER_W)
    return _embed_norm(table, idx)
